# Initial kernel scaffold; baseline (speedup 1.0000x reference)
import functools

import jax
import jax.numpy as jnp
from jax import lax
from jax.experimental import pallas as pl
from jax.experimental.pallas import tpu as pltpu
from jax.experimental.pallas import tpu_sc as plsc

N_NODES = 10000
N_EDGES = 320000
D_H = 128

NC = 2
NS = 16
NW = NC * NS
CHUNK = 128
CPW = 79
EPW = CPW * CHUNK
E_PAD = NW * EPW
ROWS_PER_TILE = 640
N_PAD = NS * ROWS_PER_TILE

_MESH = dict(core_axis_name="c", subcore_axis_name="s",
             num_cores=NC, num_subcores=NS)


def _make_sc_agg(d_feat, with_deg):
  out_type = [jax.ShapeDtypeStruct((NC, N_PAD, d_feat), jnp.float32)]
  scratch = [
      pltpu.VMEM((CPW, CHUNK), jnp.int32),
      pltpu.VMEM((CPW, CHUNK), jnp.int32),
      pltpu.VMEM((CHUNK, d_feat), jnp.float32),
      pltpu.VMEM_SHARED((N_PAD, d_feat), jnp.float32),
      pltpu.SemaphoreType.DMA,
  ]
  if with_deg:
    out_type.append(jax.ShapeDtypeStruct((NC, N_PAD, 16), jnp.float32))
    scratch.append(pltpu.VMEM((CHUNK, 16), jnp.float32))
    scratch.append(pltpu.VMEM_SHARED((N_PAD, 16), jnp.float32))

  def body(*refs):
    if with_deg:
      (x_hbm, src_hbm, dst_hbm, z_hbm, z16_hbm, ones_hbm,
       out_hbm, deg_hbm,
       src_v, dst_v, rows_v, acc_sh, sem, ones_v, deg_sh) = refs
    else:
      (x_hbm, src_hbm, dst_hbm, z_hbm,
       out_hbm,
       src_v, dst_v, rows_v, acc_sh, sem) = refs

    cid = lax.axis_index("c")
    sid = lax.axis_index("s")
    wid = sid * NC + cid
    base = sid * ROWS_PER_TILE

    for k in range(ROWS_PER_TILE // CHUNK):
      pltpu.sync_copy(z_hbm, acc_sh.at[pl.ds(base + k * CHUNK, CHUNK)])
    if with_deg:
      pltpu.sync_copy(z16_hbm, deg_sh.at[pl.ds(base, ROWS_PER_TILE)])
      pltpu.sync_copy(ones_hbm, ones_v)

    pltpu.sync_copy(src_hbm.at[wid], src_v)
    pltpu.sync_copy(dst_hbm.at[wid], dst_v)

    plsc.subcore_barrier()

    @pl.loop(0, CPW)
    def _chunk(j):
      pltpu.async_copy(x_hbm.at[src_v.at[j]], rows_v, sem).wait()
      pltpu.sync_copy(rows_v, acc_sh.at[dst_v.at[j]], add=True)
      if with_deg:
        pltpu.sync_copy(ones_v, deg_sh.at[dst_v.at[j]], add=True)

    plsc.subcore_barrier()

    pltpu.sync_copy(acc_sh.at[pl.ds(base, ROWS_PER_TILE)],
                    out_hbm.at[cid, pl.ds(base, ROWS_PER_TILE)])
    if with_deg:
      pltpu.sync_copy(deg_sh.at[pl.ds(base, ROWS_PER_TILE)],
                      deg_hbm.at[cid, pl.ds(base, ROWS_PER_TILE)])

  mesh = plsc.VectorSubcoreMesh(**_MESH)
  return pl.kernel(body, out_type=out_type, mesh=mesh, scratch_types=scratch)


def _make_tc_layer(d_out, relu):
  R = 1024
  grid = (N_PAD // R,)

  def body(x_ref, agg_ref, deg_ref, ws_ref, wn_ref, b_ref, o_ref):
    deg = deg_ref[0, :, 0:1] + deg_ref[1, :, 0:1]
    inv = 1.0 / jnp.maximum(deg, 1.0)
    neigh = (agg_ref[0] + agg_ref[1]) * inv
    y = (jnp.dot(x_ref[...], ws_ref[...], preferred_element_type=jnp.float32)
         + jnp.dot(neigh, wn_ref[...], preferred_element_type=jnp.float32)
         + b_ref[...])
    o_ref[...] = jnp.maximum(y, 0.0) if relu else y

  return pl.pallas_call(
      body,
      grid=grid,
      in_specs=[
          pl.BlockSpec((R, D_H), lambda i: (i, 0)),
          pl.BlockSpec((NC, R, D_H), lambda i: (0, i, 0)),
          pl.BlockSpec((NC, R, 16), lambda i: (0, i, 0)),
          pl.BlockSpec((D_H, d_out), lambda i: (0, 0)),
          pl.BlockSpec((D_H, d_out), lambda i: (0, 0)),
          pl.BlockSpec((1, d_out), lambda i: (0, 0)),
      ],
      out_specs=pl.BlockSpec((R, d_out), lambda i: (i, 0)),
      out_shape=jax.ShapeDtypeStruct((N_PAD, d_out), jnp.float32),
  )


@jax.jit
def kernel(inputs, Ws0, Wn0, b0, Ws1, Wn1, b1, Ws2, Wn2, b2, edge_index):
  src = edge_index[0]
  dst = edge_index[1]
  pad = E_PAD - N_EDGES
  src_p = jnp.concatenate([src, jnp.zeros((pad,), jnp.int32)])
  dst_p = jnp.concatenate([dst, jnp.full((pad,), N_NODES, jnp.int32)])
  src_r = src_p.reshape(NW, CPW, CHUNK)
  dst_r = dst_p.reshape(NW, CPW, CHUNK)

  z128 = jnp.zeros((CHUNK, D_H), jnp.float32)
  z16 = jnp.zeros((ROWS_PER_TILE, 16), jnp.float32)
  ones16 = jnp.ones((CHUNK, 16), jnp.float32)

  x_pad = jnp.pad(inputs, ((0, N_PAD - N_NODES), (0, 0)))

  agg_deg = _make_sc_agg(D_H, True)
  agg = _make_sc_agg(D_H, False)
  tc_h = _make_tc_layer(D_H, True)
  tc_out = _make_tc_layer(40, False)

  a0, deg = agg_deg(x_pad, src_r, dst_r, z128, z16, ones16)
  h1 = tc_h(x_pad, a0, deg, Ws0, Wn0, b0.reshape(1, D_H))

  a1 = agg(h1, src_r, dst_r, z128)
  h2 = tc_h(h1, a1, deg, Ws1, Wn1, b1.reshape(1, D_H))

  a2 = agg(h2, src_r, dst_r, z128)
  out = tc_out(h2, a2, deg, Ws2, Wn2, b2.reshape(1, 40))
  return out[:N_NODES]


# speedup vs baseline: 3.1410x; 3.1410x over previous
import functools

import jax
import jax.numpy as jnp
from jax import lax
from jax.experimental import pallas as pl
from jax.experimental.pallas import tpu as pltpu
from jax.experimental.pallas import tpu_sc as plsc

N_NODES = 10000
N_EDGES = 320000
D_H = 128

NC = 2
NS = 16
NW = NC * NS
CHUNK = 128
KB = 8
CPW = 80
EPW = CPW * CHUNK
E_PAD = NW * EPW
ROWS_PER_TILE = 640
N_PAD = NS * ROWS_PER_TILE

_MESH = dict(core_axis_name="c", subcore_axis_name="s",
             num_cores=NC, num_subcores=NS)


def _zero_acc(z_hbm, acc_sh, base):
  for k in range(ROWS_PER_TILE // CHUNK):
    pltpu.sync_copy(z_hbm, acc_sh.at[pl.ds(base + k * CHUNK, CHUNK)])


def _make_sc_agg():
  out_type = [jax.ShapeDtypeStruct((NC, N_PAD, D_H), jnp.float32)]
  scratch = [
      pltpu.VMEM((KB, CHUNK), jnp.int32),
      pltpu.VMEM((KB, CHUNK), jnp.int32),
      pltpu.VMEM((CHUNK, D_H), jnp.float32),
      pltpu.VMEM_SHARED((N_PAD, D_H), jnp.float32),
      pltpu.SemaphoreType.DMA,
  ]

  def body(x_hbm, src_hbm, dst_hbm, z_hbm, out_hbm,
           src_v, dst_v, rows_v, acc_sh, sem):
    cid = lax.axis_index("c")
    sid = lax.axis_index("s")
    wid = sid * NC + cid
    base = sid * ROWS_PER_TILE

    _zero_acc(z_hbm, acc_sh, base)
    plsc.subcore_barrier()

    @pl.loop(0, CPW // KB)
    def _group(g):
      pltpu.sync_copy(src_hbm.at[wid, pl.ds(g * KB, KB)], src_v)
      pltpu.sync_copy(dst_hbm.at[wid, pl.ds(g * KB, KB)], dst_v)
      for j in range(KB):
        pltpu.async_copy(x_hbm.at[src_v.at[j]], rows_v, sem).wait()
        pltpu.sync_copy(rows_v, acc_sh.at[dst_v.at[j]], add=True)

    plsc.subcore_barrier()
    pltpu.sync_copy(acc_sh.at[pl.ds(base, ROWS_PER_TILE)],
                    out_hbm.at[cid, pl.ds(base, ROWS_PER_TILE)])

  mesh = plsc.VectorSubcoreMesh(**_MESH)
  return pl.kernel(body, out_type=out_type, mesh=mesh, scratch_types=scratch)


def _make_sc_deg():
  out_type = [jax.ShapeDtypeStruct((NC, N_PAD, D_H), jnp.float32)]
  scratch = [
      pltpu.VMEM((KB, CHUNK), jnp.int32),
      pltpu.VMEM((CHUNK, D_H), jnp.float32),
      pltpu.VMEM_SHARED((N_PAD, D_H), jnp.float32),
  ]

  def body(dst_hbm, ones_hbm, z_hbm, out_hbm, dst_v, ones_v, acc_sh):
    cid = lax.axis_index("c")
    sid = lax.axis_index("s")
    wid = sid * NC + cid
    base = sid * ROWS_PER_TILE

    _zero_acc(z_hbm, acc_sh, base)
    pltpu.sync_copy(ones_hbm, ones_v)
    plsc.subcore_barrier()

    @pl.loop(0, CPW // KB)
    def _group(g):
      pltpu.sync_copy(dst_hbm.at[wid, pl.ds(g * KB, KB)], dst_v)
      for j in range(KB):
        pltpu.sync_copy(ones_v, acc_sh.at[dst_v.at[j]], add=True)

    plsc.subcore_barrier()
    pltpu.sync_copy(acc_sh.at[pl.ds(base, ROWS_PER_TILE)],
                    out_hbm.at[cid, pl.ds(base, ROWS_PER_TILE)])

  mesh = plsc.VectorSubcoreMesh(**_MESH)
  return pl.kernel(body, out_type=out_type, mesh=mesh, scratch_types=scratch)


def _make_tc_layer(d_out, relu):
  R = 1024
  grid = (N_PAD // R,)

  def body(x_ref, agg_ref, deg_ref, ws_ref, wn_ref, b_ref, o_ref):
    deg = deg_ref[0, :, 0:1] + deg_ref[1, :, 0:1]
    inv = 1.0 / jnp.maximum(deg, 1.0)
    neigh = (agg_ref[0] + agg_ref[1]) * inv
    y = (jnp.dot(x_ref[...], ws_ref[...], preferred_element_type=jnp.float32)
         + jnp.dot(neigh, wn_ref[...], preferred_element_type=jnp.float32)
         + b_ref[...])
    o_ref[...] = jnp.maximum(y, 0.0) if relu else y

  return pl.pallas_call(
      body,
      grid=grid,
      in_specs=[
          pl.BlockSpec((R, D_H), lambda i: (i, 0)),
          pl.BlockSpec((NC, R, D_H), lambda i: (0, i, 0)),
          pl.BlockSpec((NC, R, D_H), lambda i: (0, i, 0)),
          pl.BlockSpec((D_H, d_out), lambda i: (0, 0)),
          pl.BlockSpec((D_H, d_out), lambda i: (0, 0)),
          pl.BlockSpec((1, d_out), lambda i: (0, 0)),
      ],
      out_specs=pl.BlockSpec((R, d_out), lambda i: (i, 0)),
      out_shape=jax.ShapeDtypeStruct((N_PAD, d_out), jnp.float32),
  )


@jax.jit
def kernel(inputs, Ws0, Wn0, b0, Ws1, Wn1, b1, Ws2, Wn2, b2, edge_index):
  src = edge_index[0]
  dst = edge_index[1]
  pad = E_PAD - N_EDGES
  src_p = jnp.concatenate([src, jnp.zeros((pad,), jnp.int32)])
  dst_p = jnp.concatenate([dst, jnp.full((pad,), N_NODES, jnp.int32)])
  src_r = src_p.reshape(NW, CPW, CHUNK)
  dst_r = dst_p.reshape(NW, CPW, CHUNK)

  z128 = jnp.zeros((CHUNK, D_H), jnp.float32)
  ones128 = jnp.ones((CHUNK, D_H), jnp.float32)

  x_pad = jnp.pad(inputs, ((0, N_PAD - N_NODES), (0, 0)))

  agg = _make_sc_agg()
  degk = _make_sc_deg()
  tc_h = _make_tc_layer(D_H, True)
  tc_out = _make_tc_layer(40, False)

  [deg] = degk(dst_r, ones128, z128)
  [a0] = agg(x_pad, src_r, dst_r, z128)
  h1 = tc_h(x_pad, a0, deg, Ws0, Wn0, b0.reshape(1, D_H))

  [a1] = agg(h1, src_r, dst_r, z128)
  h2 = tc_h(h1, a1, deg, Ws1, Wn1, b1.reshape(1, D_H))

  [a2] = agg(h2, src_r, dst_r, z128)
  out = tc_out(h2, a2, deg, Ws2, Wn2, b2.reshape(1, 40))
  return out[:N_NODES]


# speedup vs baseline: 3.3805x; 1.0762x over previous
import functools

import jax
import jax.numpy as jnp
from jax import lax
from jax.experimental import pallas as pl
from jax.experimental.pallas import tpu as pltpu
from jax.experimental.pallas import tpu_sc as plsc

N_NODES = 10000
N_EDGES = 320000
D_H = 128

NC = 2
NS = 16
NW = NC * NS
CHUNK = 128
KB = 8
CPW = 80
EPW = CPW * CHUNK
E_PAD = NW * EPW
ROWS_PER_TILE = 640
N_PAD = NS * ROWS_PER_TILE

_MESH = dict(core_axis_name="c", subcore_axis_name="s",
             num_cores=NC, num_subcores=NS)


def _zero_acc(z_hbm, acc_sh, base):
  for k in range(ROWS_PER_TILE // CHUNK):
    pltpu.sync_copy(z_hbm, acc_sh.at[pl.ds(base + k * CHUNK, CHUNK)])


def _make_sc_agg():
  out_type = [jax.ShapeDtypeStruct((NC, N_PAD, D_H), jnp.float32)]
  scratch = [
      pltpu.VMEM((KB, CHUNK), jnp.int32),
      pltpu.VMEM((KB, CHUNK), jnp.int32),
      pltpu.VMEM((CHUNK, D_H), jnp.float32),
      pltpu.VMEM((CHUNK, D_H), jnp.float32),
      pltpu.VMEM_SHARED((N_PAD, D_H), jnp.float32),
      pltpu.SemaphoreType.DMA,
      pltpu.SemaphoreType.DMA,
  ]

  def body(x_hbm, src_hbm, dst_hbm, z_hbm, out_hbm,
           src_v, dst_v, rows_a, rows_b, acc_sh, sem_a, sem_b):
    cid = lax.axis_index("c")
    sid = lax.axis_index("s")
    wid = sid * NC + cid
    base = sid * ROWS_PER_TILE

    _zero_acc(z_hbm, acc_sh, base)
    plsc.subcore_barrier()

    bufs = [(rows_a, sem_a), (rows_b, sem_b)]

    @pl.loop(0, CPW // KB)
    def _group(g):
      pltpu.sync_copy(src_hbm.at[wid, pl.ds(g * KB, KB)], src_v)
      pltpu.sync_copy(dst_hbm.at[wid, pl.ds(g * KB, KB)], dst_v)
      pltpu.async_copy(x_hbm.at[src_v.at[0]], rows_a, sem_a)
      for j in range(KB):
        rows_j, sem_j = bufs[j % 2]
        pltpu.make_async_copy(x_hbm.at[src_v.at[j]], rows_j, sem_j).wait()
        if j + 1 < KB:
          rows_n, sem_n = bufs[(j + 1) % 2]
          pltpu.async_copy(x_hbm.at[src_v.at[j + 1]], rows_n, sem_n)
        pltpu.sync_copy(rows_j, acc_sh.at[dst_v.at[j]], add=True)

    plsc.subcore_barrier()
    pltpu.sync_copy(acc_sh.at[pl.ds(base, ROWS_PER_TILE)],
                    out_hbm.at[cid, pl.ds(base, ROWS_PER_TILE)])

  mesh = plsc.VectorSubcoreMesh(**_MESH)
  return pl.kernel(body, out_type=out_type, mesh=mesh, scratch_types=scratch)


def _make_sc_deg():
  out_type = [jax.ShapeDtypeStruct((NC, N_PAD, D_H), jnp.float32)]
  scratch = [
      pltpu.VMEM((KB, CHUNK), jnp.int32),
      pltpu.VMEM((CHUNK, D_H), jnp.float32),
      pltpu.VMEM_SHARED((N_PAD, D_H), jnp.float32),
  ]

  def body(dst_hbm, ones_hbm, z_hbm, out_hbm, dst_v, ones_v, acc_sh):
    cid = lax.axis_index("c")
    sid = lax.axis_index("s")
    wid = sid * NC + cid
    base = sid * ROWS_PER_TILE

    _zero_acc(z_hbm, acc_sh, base)
    pltpu.sync_copy(ones_hbm, ones_v)
    plsc.subcore_barrier()

    @pl.loop(0, CPW // KB)
    def _group(g):
      pltpu.sync_copy(dst_hbm.at[wid, pl.ds(g * KB, KB)], dst_v)
      for j in range(KB):
        pltpu.sync_copy(ones_v, acc_sh.at[dst_v.at[j]], add=True)

    plsc.subcore_barrier()
    pltpu.sync_copy(acc_sh.at[pl.ds(base, ROWS_PER_TILE)],
                    out_hbm.at[cid, pl.ds(base, ROWS_PER_TILE)])

  mesh = plsc.VectorSubcoreMesh(**_MESH)
  return pl.kernel(body, out_type=out_type, mesh=mesh, scratch_types=scratch)


def _make_tc_layer(d_out, relu):
  R = 1024
  grid = (N_PAD // R,)

  def body(x_ref, agg_ref, deg_ref, ws_ref, wn_ref, b_ref, o_ref):
    deg = deg_ref[0, :, 0:1] + deg_ref[1, :, 0:1]
    inv = 1.0 / jnp.maximum(deg, 1.0)
    neigh = (agg_ref[0] + agg_ref[1]) * inv
    y = (jnp.dot(x_ref[...], ws_ref[...], preferred_element_type=jnp.float32)
         + jnp.dot(neigh, wn_ref[...], preferred_element_type=jnp.float32)
         + b_ref[...])
    o_ref[...] = jnp.maximum(y, 0.0) if relu else y

  return pl.pallas_call(
      body,
      grid=grid,
      in_specs=[
          pl.BlockSpec((R, D_H), lambda i: (i, 0)),
          pl.BlockSpec((NC, R, D_H), lambda i: (0, i, 0)),
          pl.BlockSpec((NC, R, D_H), lambda i: (0, i, 0)),
          pl.BlockSpec((D_H, d_out), lambda i: (0, 0)),
          pl.BlockSpec((D_H, d_out), lambda i: (0, 0)),
          pl.BlockSpec((1, d_out), lambda i: (0, 0)),
      ],
      out_specs=pl.BlockSpec((R, d_out), lambda i: (i, 0)),
      out_shape=jax.ShapeDtypeStruct((N_PAD, d_out), jnp.float32),
  )


@jax.jit
def kernel(inputs, Ws0, Wn0, b0, Ws1, Wn1, b1, Ws2, Wn2, b2, edge_index):
  src = edge_index[0]
  dst = edge_index[1]
  pad = E_PAD - N_EDGES
  src_p = jnp.concatenate([src, jnp.zeros((pad,), jnp.int32)])
  dst_p = jnp.concatenate([dst, jnp.full((pad,), N_NODES, jnp.int32)])
  src_r = src_p.reshape(NW, CPW, CHUNK)
  dst_r = dst_p.reshape(NW, CPW, CHUNK)

  z128 = jnp.zeros((CHUNK, D_H), jnp.float32)
  ones128 = jnp.ones((CHUNK, D_H), jnp.float32)

  x_pad = jnp.pad(inputs, ((0, N_PAD - N_NODES), (0, 0)))

  agg = _make_sc_agg()
  degk = _make_sc_deg()
  tc_h = _make_tc_layer(D_H, True)
  tc_out = _make_tc_layer(40, False)

  [deg] = degk(dst_r, ones128, z128)
  [a0] = agg(x_pad, src_r, dst_r, z128)
  h1 = tc_h(x_pad, a0, deg, Ws0, Wn0, b0.reshape(1, D_H))

  [a1] = agg(h1, src_r, dst_r, z128)
  h2 = tc_h(h1, a1, deg, Ws1, Wn1, b1.reshape(1, D_H))

  [a2] = agg(h2, src_r, dst_r, z128)
  out = tc_out(h2, a2, deg, Ws2, Wn2, b2.reshape(1, 40))
  return out[:N_NODES]


# speedup vs baseline: 3.3822x; 1.0005x over previous
import functools

import jax
import jax.numpy as jnp
from jax import lax
from jax.experimental import pallas as pl
from jax.experimental.pallas import tpu as pltpu
from jax.experimental.pallas import tpu_sc as plsc

N_NODES = 10000
N_EDGES = 320000
D_H = 128

NC = 2
NS = 16
NW = NC * NS
CHUNK = 64
KB = 40
CPW = 160
NBUF = 4
EPW = CPW * CHUNK
E_PAD = NW * EPW
ROWS_PER_TILE = 640
N_PAD = NS * ROWS_PER_TILE

_MESH = dict(core_axis_name="c", subcore_axis_name="s",
             num_cores=NC, num_subcores=NS)


def _zero_acc(z_hbm, acc_sh, base):
  for k in range(ROWS_PER_TILE // CHUNK):
    pltpu.sync_copy(z_hbm, acc_sh.at[pl.ds(base + k * CHUNK, CHUNK)])


def _make_sc_agg():
  out_type = [jax.ShapeDtypeStruct((NC, N_PAD, D_H), jnp.float32)]
  scratch = [
      pltpu.VMEM((KB, CHUNK), jnp.int32),
      pltpu.VMEM((KB, CHUNK), jnp.int32),
      [pltpu.VMEM((CHUNK, D_H), jnp.float32) for _ in range(NBUF)],
      pltpu.VMEM_SHARED((N_PAD, D_H), jnp.float32),
      [pltpu.SemaphoreType.DMA for _ in range(NBUF)],
      [pltpu.SemaphoreType.DMA for _ in range(NBUF)],
  ]

  def body(x_hbm, src_hbm, dst_hbm, z_hbm, out_hbm,
           src_v, dst_v, rows, acc_sh, sem_g, sem_s):
    cid = lax.axis_index("c")
    sid = lax.axis_index("s")
    wid = sid * NC + cid
    base = sid * ROWS_PER_TILE

    _zero_acc(z_hbm, acc_sh, base)
    plsc.subcore_barrier()

    @pl.loop(0, CPW // KB)
    def _group(g):
      pltpu.sync_copy(src_hbm.at[wid, pl.ds(g * KB, KB)], src_v)
      pltpu.sync_copy(dst_hbm.at[wid, pl.ds(g * KB, KB)], dst_v)

      @pl.loop(0, KB // NBUF)
      def _quad(t):
        gds = [pltpu.async_copy(x_hbm.at[src_v.at[t * NBUF + u]],
                                rows[u], sem_g[u])
               for u in range(NBUF)]
        for d in gds:
          d.wait()
        sds = [pltpu.async_copy(rows[u], acc_sh.at[dst_v.at[t * NBUF + u]],
                                sem_s[u], add=True)
               for u in range(NBUF)]
        for d in sds:
          d.wait()

    plsc.subcore_barrier()
    pltpu.sync_copy(acc_sh.at[pl.ds(base, ROWS_PER_TILE)],
                    out_hbm.at[cid, pl.ds(base, ROWS_PER_TILE)])

  mesh = plsc.VectorSubcoreMesh(**_MESH)
  return pl.kernel(body, out_type=out_type, mesh=mesh, scratch_types=scratch)


def _make_sc_deg():
  out_type = [jax.ShapeDtypeStruct((NC, N_PAD, D_H), jnp.float32)]
  scratch = [
      pltpu.VMEM((KB, CHUNK), jnp.int32),
      pltpu.VMEM((CHUNK, D_H), jnp.float32),
      pltpu.VMEM_SHARED((N_PAD, D_H), jnp.float32),
      [pltpu.SemaphoreType.DMA for _ in range(NBUF)],
  ]

  def body(dst_hbm, ones_hbm, z_hbm, out_hbm, dst_v, ones_v, acc_sh, sem_s):
    cid = lax.axis_index("c")
    sid = lax.axis_index("s")
    wid = sid * NC + cid
    base = sid * ROWS_PER_TILE

    _zero_acc(z_hbm, acc_sh, base)
    pltpu.sync_copy(ones_hbm, ones_v)
    plsc.subcore_barrier()

    @pl.loop(0, CPW // KB)
    def _group(g):
      pltpu.sync_copy(dst_hbm.at[wid, pl.ds(g * KB, KB)], dst_v)

      @pl.loop(0, KB // NBUF)
      def _quad(t):
        sds = [pltpu.async_copy(ones_v, acc_sh.at[dst_v.at[t * NBUF + u]],
                                sem_s[u], add=True)
               for u in range(NBUF)]
        for d in sds:
          d.wait()

    plsc.subcore_barrier()
    pltpu.sync_copy(acc_sh.at[pl.ds(base, ROWS_PER_TILE)],
                    out_hbm.at[cid, pl.ds(base, ROWS_PER_TILE)])

  mesh = plsc.VectorSubcoreMesh(**_MESH)
  return pl.kernel(body, out_type=out_type, mesh=mesh, scratch_types=scratch)


def _make_tc_layer(d_out, relu):
  R = 1024
  grid = (N_PAD // R,)

  def body(x_ref, agg_ref, deg_ref, ws_ref, wn_ref, b_ref, o_ref):
    deg = deg_ref[0, :, 0:1] + deg_ref[1, :, 0:1]
    inv = 1.0 / jnp.maximum(deg, 1.0)
    neigh = (agg_ref[0] + agg_ref[1]) * inv
    y = (jnp.dot(x_ref[...], ws_ref[...], preferred_element_type=jnp.float32)
         + jnp.dot(neigh, wn_ref[...], preferred_element_type=jnp.float32)
         + b_ref[...])
    o_ref[...] = jnp.maximum(y, 0.0) if relu else y

  return pl.pallas_call(
      body,
      grid=grid,
      in_specs=[
          pl.BlockSpec((R, D_H), lambda i: (i, 0)),
          pl.BlockSpec((NC, R, D_H), lambda i: (0, i, 0)),
          pl.BlockSpec((NC, R, D_H), lambda i: (0, i, 0)),
          pl.BlockSpec((D_H, d_out), lambda i: (0, 0)),
          pl.BlockSpec((D_H, d_out), lambda i: (0, 0)),
          pl.BlockSpec((1, d_out), lambda i: (0, 0)),
      ],
      out_specs=pl.BlockSpec((R, d_out), lambda i: (i, 0)),
      out_shape=jax.ShapeDtypeStruct((N_PAD, d_out), jnp.float32),
  )


@jax.jit
def kernel(inputs, Ws0, Wn0, b0, Ws1, Wn1, b1, Ws2, Wn2, b2, edge_index):
  src = edge_index[0]
  dst = edge_index[1]
  pad = E_PAD - N_EDGES
  src_p = jnp.concatenate([src, jnp.zeros((pad,), jnp.int32)])
  dst_p = jnp.concatenate([dst, jnp.full((pad,), N_NODES, jnp.int32)])
  src_r = src_p.reshape(NW, CPW, CHUNK)
  dst_r = dst_p.reshape(NW, CPW, CHUNK)

  z128 = jnp.zeros((CHUNK, D_H), jnp.float32)
  ones128 = jnp.ones((CHUNK, D_H), jnp.float32)

  x_pad = jnp.pad(inputs, ((0, N_PAD - N_NODES), (0, 0)))

  agg = _make_sc_agg()
  degk = _make_sc_deg()
  tc_h = _make_tc_layer(D_H, True)
  tc_out = _make_tc_layer(40, False)

  [deg] = degk(dst_r, ones128, z128)
  [a0] = agg(x_pad, src_r, dst_r, z128)
  h1 = tc_h(x_pad, a0, deg, Ws0, Wn0, b0.reshape(1, D_H))

  [a1] = agg(h1, src_r, dst_r, z128)
  h2 = tc_h(h1, a1, deg, Ws1, Wn1, b1.reshape(1, D_H))

  [a2] = agg(h2, src_r, dst_r, z128)
  out = tc_out(h2, a2, deg, Ws2, Wn2, b2.reshape(1, 40))
  return out[:N_NODES]
